# Initial kernel scaffold; baseline (speedup 1.0000x reference)
import jax
import jax.numpy as jnp
from jax.experimental import pallas as pl


def kernel(x, edge_index, edge_weight, W1_0, b1, W2_0, W2_1, b2, W3_0, W3_1, W3_2, b3):
    raise NotImplementedError("write your pallas kernel here")


# speedup vs baseline: 8.5327x; 8.5327x over previous
import functools

import jax
import jax.numpy as jnp
from jax import lax
from jax.experimental import pallas as pl
from jax.experimental.pallas import tpu as pltpu
from jax.experimental.pallas import tpu_sc as plsc

N = 10000
E = 320000
D = 128
DO3 = 1200

NC = 2
NS = 16
NW = NC * NS

EA = E // NS
EW = E // NW
WIN = 80
NWIN = EW // WIN

_MESH = plsc.VectorSubcoreMesh(core_axis_name="c", subcore_axis_name="s")


def _rsqrt16(d):
    di = lax.bitcast_convert_type(d, jnp.int32)
    yi = jnp.int32(0x5F3759DF) - lax.shift_right_logical(di, 1)
    y = lax.bitcast_convert_type(yi, jnp.float32)
    for _ in range(3):
        y = y * (1.5 - 0.5 * d * y * y)
    return y


def _off_body(row_hbm, col_hbm, w_hbm, off_hbm, deg_sh, ibuf, jbuf, wbuf, abuf):
    c = lax.axis_index("c")
    s = lax.axis_index("s")
    wid = s * NC + c

    e0 = s * EA
    pltpu.sync_copy(row_hbm.at[pl.ds(e0, EA)], ibuf)
    pltpu.sync_copy(col_hbm.at[pl.ds(e0, EA)], jbuf)
    pltpu.sync_copy(w_hbm.at[pl.ds(e0, EA)], wbuf)

    def _z(i, _):
        abuf[pl.ds(i * 16, 16)] = jnp.zeros((16,), jnp.float32)
        return _
    lax.fori_loop(0, 40, _z, 0)
    pltpu.sync_copy(abuf.at[pl.ds(0, 640)], deg_sh.at[pl.ds(s * 640, 640)])

    def _wm(i, _):
        sl = pl.ds(i * 16, 16)
        r = ibuf[sl]
        cc = jbuf[sl]
        w = wbuf[sl]
        abuf[sl] = jnp.where(r == cc, 0.0, w)
        return _
    lax.fori_loop(0, EA // 16, _wm, 0)

    plsc.subcore_barrier()
    pltpu.sync_copy(abuf, deg_sh.at[ibuf], add=True)
    plsc.subcore_barrier()

    pltpu.sync_copy(deg_sh.at[pl.ds(0, N)], abuf.at[pl.ds(0, N)])

    def _ns(i, _):
        sl = pl.ds(i * 16, 16)
        d = abuf[sl]
        y = _rsqrt16(d)
        abuf[sl] = jnp.where(d > 0.0, y, 0.0)
        return _
    lax.fori_loop(0, N // 16, _ns, 0)

    b0 = wid * EW
    pltpu.sync_copy(row_hbm.at[pl.ds(b0, EW)], ibuf.at[pl.ds(0, EW)])
    pltpu.sync_copy(col_hbm.at[pl.ds(b0, EW)], jbuf.at[pl.ds(0, EW)])
    pltpu.sync_copy(w_hbm.at[pl.ds(b0, EW)], wbuf.at[pl.ds(0, EW)])

    def _off(i, _):
        sl = pl.ds(i * 16, 16)
        r = ibuf[sl]
        cc = jbuf[sl]
        w = wbuf[sl]
        dr = plsc.load_gather(abuf, [r])
        dc = plsc.load_gather(abuf, [cc])
        v = -(dr * w * dc)
        wbuf[pl.ds(EW + i * 16, 16)] = jnp.where(r == cc, 0.0, v)
        return _
    lax.fori_loop(0, EW // 16, _off, 0)

    pltpu.sync_copy(wbuf.at[pl.ds(EW, EW)], off_hbm.at[pl.ds(b0, EW)])


_off_kernel = functools.partial(
    pl.kernel,
    out_type=jax.ShapeDtypeStruct((E,), jnp.float32),
    mesh=_MESH,
    compiler_params=pltpu.CompilerParams(needs_layout_passes=False, use_tc_tiling_on_sc=False),
    scratch_types=[
        pltpu.VMEM_SHARED((10240,), jnp.float32),
        pltpu.VMEM((EA,), jnp.int32),
        pltpu.VMEM((EA,), jnp.int32),
        pltpu.VMEM((2 * EW,), jnp.float32),
        pltpu.VMEM((2 * EW,), jnp.float32),
    ],
)(_off_body)


def _prop_body(v_hbm, row_hbm, col_hbm, off_hbm, out_hbm,
               acc_sh, gbuf, cidx, ridx, offb, zbuf):
    c = lax.axis_index("c")
    s = lax.axis_index("s")
    wid = s * NC + c

    for r in range(25):
        for f in range(8):
            zbuf[r, pl.ds(f * 16, 16)] = jnp.zeros((16,), jnp.float32)

    def _za(k, _):
        pltpu.sync_copy(zbuf, acc_sh.at[pl.ds(s * 625 + k * 25, 25)])
        return _
    lax.fori_loop(0, 25, _za, 0)

    b0 = wid * EW
    pltpu.sync_copy(row_hbm.at[pl.ds(b0, EW)], ridx)
    pltpu.sync_copy(off_hbm.at[pl.ds(b0, EW)], offb)
    plsc.subcore_barrier()

    def _win(w, _):
        wb = w * WIN
        pltpu.sync_copy(col_hbm.at[pl.ds(b0 + wb, WIN)], cidx.at[0])
        pltpu.sync_copy(v_hbm.at[ridx.at[pl.ds(wb, WIN)]], gbuf)

        def _edge(e, _2):
            p = wb + e
            splat = plsc.load_gather(offb, [jnp.full((16,), 0, jnp.int32) + p])
            for f in range(8):
                sl = pl.ds(f * 16, 16)
                gbuf[e, sl] = gbuf[e, sl] * splat
            return _2
        lax.fori_loop(0, WIN, _edge, 0)

        pltpu.sync_copy(gbuf, acc_sh.at[cidx.at[0]], add=True)
        return _
    lax.fori_loop(0, NWIN, _win, 0)

    plsc.subcore_barrier()
    pltpu.sync_copy(acc_sh.at[pl.ds(s * 625, 625)],
                    out_hbm.at[c, pl.ds(s * 625, 625)])


_prop_kernel = functools.partial(
    pl.kernel,
    out_type=jax.ShapeDtypeStruct((NC, N, D), jnp.float32),
    mesh=_MESH,
    compiler_params=pltpu.CompilerParams(needs_layout_passes=False, use_tc_tiling_on_sc=False),
    scratch_types=[
        pltpu.VMEM_SHARED((N, D), jnp.float32),
        pltpu.VMEM((WIN, D), jnp.float32),
        pltpu.VMEM((2, WIN), jnp.int32),
        pltpu.VMEM((EW,), jnp.int32),
        pltpu.VMEM((EW,), jnp.float32),
        pltpu.VMEM((25, D), jnp.float32),
    ],
)(_prop_body)


def _add_body(a_ref, b_ref, o_ref):
    o_ref[...] = a_ref[...] + b_ref[...]


def _combine_partials(p):
    return pl.pallas_call(
        _add_body,
        grid=(10,),
        in_specs=[pl.BlockSpec((N // 10, D), lambda i: (i, 0)),
                  pl.BlockSpec((N // 10, D), lambda i: (i, 0))],
        out_specs=pl.BlockSpec((N // 10, D), lambda i: (i, 0)),
        out_shape=jax.ShapeDtypeStruct((N, D), jnp.float32),
    )(p[0], p[1])


def _mm_body(x_ref, t1_ref, q0_ref, q1_ref, wa_ref, wb_ref, wc_ref, b_ref,
             o_ref):
    xb = x_ref[...]
    t1 = t1_ref[...]
    qb = q0_ref[...] + q1_ref[...]
    o = jnp.dot(xb, wa_ref[...], preferred_element_type=jnp.float32)
    o += jnp.dot(t1, wb_ref[...], preferred_element_type=jnp.float32)
    o += jnp.dot(qb, wc_ref[...], preferred_element_type=jnp.float32)
    o_ref[...] = o + b_ref[...]


def _matmul(x, t1, q0, q1, wa, wb, wc, bias):
    br = N // 10
    return pl.pallas_call(
        _mm_body,
        grid=(10,),
        in_specs=[pl.BlockSpec((br, D), lambda i: (i, 0)),
                  pl.BlockSpec((br, D), lambda i: (i, 0)),
                  pl.BlockSpec((br, D), lambda i: (i, 0)),
                  pl.BlockSpec((br, D), lambda i: (i, 0)),
                  pl.BlockSpec((D, DO3), lambda i: (0, 0)),
                  pl.BlockSpec((D, DO3), lambda i: (0, 0)),
                  pl.BlockSpec((D, DO3), lambda i: (0, 0)),
                  pl.BlockSpec((1, DO3), lambda i: (0, 0))],
        out_specs=pl.BlockSpec((br, DO3), lambda i: (i, 0)),
        out_shape=jax.ShapeDtypeStruct((N, DO3), jnp.float32),
    )(x, t1, q0, q1, wa, wb, wc, bias)


def kernel(x, edge_index, edge_weight, W1_0, b1, W2_0, W2_1, b2,
           W3_0, W3_1, W3_2, b3):
    row = edge_index[0]
    col = edge_index[1]

    off = _off_kernel(row, col, edge_weight)
    p1 = _prop_kernel(x, row, col, off)
    t1 = _combine_partials(p1)
    p2 = _prop_kernel(t1, row, col, off)

    z4 = jnp.zeros((D, 400), jnp.float32)
    wa = jnp.concatenate([W1_0, W2_0, W3_0 - W3_2], axis=1)
    wb = jnp.concatenate([z4, W2_1, W3_1], axis=1)
    wc = jnp.concatenate([z4, z4, 2.0 * W3_2], axis=1)
    bias = jnp.concatenate([b1, b2, b3])[None, :]
    return _matmul(x, t1, p2[0], p2[1], wa, wb, wc, bias)


# speedup vs baseline: 14.1507x; 1.6584x over previous
import functools

import jax
import jax.numpy as jnp
from jax import lax
from jax.experimental import pallas as pl
from jax.experimental.pallas import tpu as pltpu
from jax.experimental.pallas import tpu_sc as plsc

N = 10000
E = 320000
D = 128
DO3 = 1200

NC = 2
NS = 16
NW = NC * NS

EA = E // NS
EW = E // NW
WIN = 80
NWIN = EW // WIN

_MESH = plsc.VectorSubcoreMesh(core_axis_name="c", subcore_axis_name="s")


def _rsqrt16(d):
    di = lax.bitcast_convert_type(d, jnp.int32)
    yi = jnp.int32(0x5F3759DF) - lax.shift_right_logical(di, 1)
    y = lax.bitcast_convert_type(yi, jnp.float32)
    for _ in range(3):
        y = y * (1.5 - 0.5 * d * y * y)
    return y


def _off_body(row_hbm, col_hbm, w_hbm, off_hbm, deg_sh, ibuf, jbuf, wbuf, abuf):
    c = lax.axis_index("c")
    s = lax.axis_index("s")
    wid = s * NC + c

    e0 = s * EA
    pltpu.sync_copy(row_hbm.at[pl.ds(e0, EA)], ibuf)
    pltpu.sync_copy(col_hbm.at[pl.ds(e0, EA)], jbuf)
    pltpu.sync_copy(w_hbm.at[pl.ds(e0, EA)], wbuf)

    def _z(i, _):
        abuf[pl.ds(i * 16, 16)] = jnp.zeros((16,), jnp.float32)
        return _
    lax.fori_loop(0, 40, _z, 0)
    pltpu.sync_copy(abuf.at[pl.ds(0, 640)], deg_sh.at[pl.ds(s * 640, 640)])

    def _wm(i, _):
        sl = pl.ds(i * 16, 16)
        r = ibuf[sl]
        cc = jbuf[sl]
        w = wbuf[sl]
        abuf[sl] = jnp.where(r == cc, 0.0, w)
        return _
    lax.fori_loop(0, EA // 16, _wm, 0)

    plsc.subcore_barrier()
    pltpu.sync_copy(abuf, deg_sh.at[ibuf], add=True)
    plsc.subcore_barrier()

    pltpu.sync_copy(deg_sh.at[pl.ds(0, N)], abuf.at[pl.ds(0, N)])

    def _ns(i, _):
        sl = pl.ds(i * 16, 16)
        d = abuf[sl]
        y = _rsqrt16(d)
        abuf[sl] = jnp.where(d > 0.0, y, 0.0)
        return _
    lax.fori_loop(0, N // 16, _ns, 0)

    b0 = wid * EW
    pltpu.sync_copy(row_hbm.at[pl.ds(b0, EW)], ibuf.at[pl.ds(0, EW)])
    pltpu.sync_copy(col_hbm.at[pl.ds(b0, EW)], jbuf.at[pl.ds(0, EW)])
    pltpu.sync_copy(w_hbm.at[pl.ds(b0, EW)], wbuf.at[pl.ds(0, EW)])

    def _off(i, _):
        sl = pl.ds(i * 16, 16)
        r = ibuf[sl]
        cc = jbuf[sl]
        w = wbuf[sl]
        dr = plsc.load_gather(abuf, [r])
        dc = plsc.load_gather(abuf, [cc])
        v = -(dr * w * dc)
        wbuf[pl.ds(EW + i * 16, 16)] = jnp.where(r == cc, 0.0, v)
        return _
    lax.fori_loop(0, EW // 16, _off, 0)

    pltpu.sync_copy(wbuf.at[pl.ds(EW, EW)], off_hbm.at[pl.ds(b0, EW)])


_off_kernel = functools.partial(
    pl.kernel,
    out_type=jax.ShapeDtypeStruct((E,), jnp.float32),
    mesh=_MESH,
    compiler_params=pltpu.CompilerParams(needs_layout_passes=False, use_tc_tiling_on_sc=False),
    scratch_types=[
        pltpu.VMEM_SHARED((10240,), jnp.float32),
        pltpu.VMEM((EA,), jnp.int32),
        pltpu.VMEM((EA,), jnp.int32),
        pltpu.VMEM((2 * EW,), jnp.float32),
        pltpu.VMEM((2 * EW,), jnp.float32),
    ],
)(_off_body)


def _prop_body(v_hbm, row_hbm, col_hbm, off_hbm, out_hbm,
               acc_sh, gbuf, cidx, ridx, offb, zbuf,
               gsem0, gsem1, ssem0, ssem1, csem0, csem1):
    c = lax.axis_index("c")
    s = lax.axis_index("s")
    wid = s * NC + c

    for r in range(8):
        for f in range(8):
            zbuf[r, pl.ds(f * 16, 16)] = jnp.zeros((16,), jnp.float32)

    def _za(k, _):
        pltpu.sync_copy(zbuf, acc_sh.at[pl.ds(s * 625 + k * 8, 8)])
        return _
    lax.fori_loop(0, 78, _za, 0)
    pltpu.sync_copy(zbuf.at[pl.ds(0, 1)], acc_sh.at[pl.ds(s * 625 + 624, 1)])

    b0 = wid * EW
    pltpu.sync_copy(row_hbm.at[pl.ds(b0, EW)], ridx)
    pltpu.sync_copy(off_hbm.at[pl.ds(b0, EW)], offb)
    plsc.subcore_barrier()

    gsems = (gsem0, gsem1)
    ssems = (ssem0, ssem1)
    csems = (csem0, csem1)

    def _start_gather(w, b):
        pltpu.async_copy(v_hbm.at[ridx.at[pl.ds(w * WIN, WIN)]],
                         gbuf.at[b], gsems[b])
        pltpu.async_copy(col_hbm.at[pl.ds(b0 + w * WIN, WIN)],
                         cidx.at[b], csems[b])

    def _wait_cidx(b):
        pltpu.make_async_copy(col_hbm.at[pl.ds(0, WIN)], cidx.at[b],
                              csems[b]).wait()

    def _scale(w, b):
        def _edge4(e4, _2):
            for u in range(4):
                e = e4 * 4 + u
                p = w * WIN + e
                splat = plsc.load_gather(
                    offb, [jnp.full((16,), 0, jnp.int32) + p])
                for f in range(8):
                    sl = pl.ds(f * 16, 16)
                    gbuf[b, e, sl] = gbuf[b, e, sl] * splat
            return _2
        lax.fori_loop(0, WIN // 4, _edge4, 0)

    def _start_scatter(w, b):
        pltpu.async_copy(gbuf.at[b], acc_sh.at[cidx.at[b]], ssems[b],
                         add=True)

    def _wait_gather(b):
        pltpu.make_async_copy(v_hbm.at[pl.ds(0, WIN)], gbuf.at[b],
                              gsems[b]).wait()

    def _wait_scatter(b):
        pltpu.make_async_copy(v_hbm.at[pl.ds(0, WIN)], gbuf.at[b],
                              ssems[b]).wait()

    KH = NWIN // 2
    _start_gather(0, 0)
    _start_gather(1, 1)

    def _pair(k, _):
        w0 = k * 2
        _wait_gather(0)
        _scale(w0, 0)
        _wait_cidx(0)
        _start_scatter(w0, 0)
        _wait_gather(1)
        _scale(w0 + 1, 1)
        _wait_scatter(0)

        @pl.when(k < KH - 1)
        def _pf0():
            _start_gather(w0 + 2, 0)
        _wait_cidx(1)
        _start_scatter(w0 + 1, 1)
        _wait_scatter(1)

        @pl.when(k < KH - 1)
        def _pf1():
            _start_gather(w0 + 3, 1)
        return _
    lax.fori_loop(0, KH, _pair, 0)

    wt = NWIN - 1
    _start_gather(wt, 0)
    _wait_gather(0)
    _scale(wt, 0)
    _wait_cidx(0)
    _start_scatter(wt, 0)
    _wait_scatter(0)

    plsc.subcore_barrier()
    pltpu.sync_copy(acc_sh.at[pl.ds(s * 625, 625)],
                    out_hbm.at[c, pl.ds(s * 625, 625)])


_prop_kernel = functools.partial(
    pl.kernel,
    out_type=jax.ShapeDtypeStruct((NC, N, D), jnp.float32),
    mesh=_MESH,
    compiler_params=pltpu.CompilerParams(needs_layout_passes=False, use_tc_tiling_on_sc=False),
    scratch_types=[
        pltpu.VMEM_SHARED((N, D), jnp.float32),
        pltpu.VMEM((2, WIN, D), jnp.float32),
        pltpu.VMEM((2, WIN), jnp.int32),
        pltpu.VMEM((EW,), jnp.int32),
        pltpu.VMEM((EW,), jnp.float32),
        pltpu.VMEM((8, D), jnp.float32),
        pltpu.SemaphoreType.DMA,
        pltpu.SemaphoreType.DMA,
        pltpu.SemaphoreType.DMA,
        pltpu.SemaphoreType.DMA,
        pltpu.SemaphoreType.DMA,
        pltpu.SemaphoreType.DMA,
    ],
)(_prop_body)


def _add_body(a_ref, b_ref, o_ref):
    o_ref[...] = a_ref[...] + b_ref[...]


def _combine_partials(p):
    return pl.pallas_call(
        _add_body,
        grid=(10,),
        in_specs=[pl.BlockSpec((N // 10, D), lambda i: (i, 0)),
                  pl.BlockSpec((N // 10, D), lambda i: (i, 0))],
        out_specs=pl.BlockSpec((N // 10, D), lambda i: (i, 0)),
        out_shape=jax.ShapeDtypeStruct((N, D), jnp.float32),
    )(p[0], p[1])


def _mm_body(x_ref, t1_ref, q0_ref, q1_ref, wa_ref, wb_ref, wc_ref, b_ref,
             o_ref):
    xb = x_ref[...]
    t1 = t1_ref[...]
    qb = q0_ref[...] + q1_ref[...]
    o = jnp.dot(xb, wa_ref[...], preferred_element_type=jnp.float32)
    o += jnp.dot(t1, wb_ref[...], preferred_element_type=jnp.float32)
    o += jnp.dot(qb, wc_ref[...], preferred_element_type=jnp.float32)
    o_ref[...] = o + b_ref[...]


def _matmul(x, t1, q0, q1, wa, wb, wc, bias):
    br = N // 10
    return pl.pallas_call(
        _mm_body,
        grid=(10,),
        in_specs=[pl.BlockSpec((br, D), lambda i: (i, 0)),
                  pl.BlockSpec((br, D), lambda i: (i, 0)),
                  pl.BlockSpec((br, D), lambda i: (i, 0)),
                  pl.BlockSpec((br, D), lambda i: (i, 0)),
                  pl.BlockSpec((D, DO3), lambda i: (0, 0)),
                  pl.BlockSpec((D, DO3), lambda i: (0, 0)),
                  pl.BlockSpec((D, DO3), lambda i: (0, 0)),
                  pl.BlockSpec((1, DO3), lambda i: (0, 0))],
        out_specs=pl.BlockSpec((br, DO3), lambda i: (i, 0)),
        out_shape=jax.ShapeDtypeStruct((N, DO3), jnp.float32),
    )(x, t1, q0, q1, wa, wb, wc, bias)


def kernel(x, edge_index, edge_weight, W1_0, b1, W2_0, W2_1, b2,
           W3_0, W3_1, W3_2, b3):
    row = edge_index[0]
    col = edge_index[1]

    off = _off_kernel(row, col, edge_weight)
    p1 = _prop_kernel(x, row, col, off)
    t1 = _combine_partials(p1)
    p2 = _prop_kernel(t1, row, col, off)

    z4 = jnp.zeros((D, 400), jnp.float32)
    wa = jnp.concatenate([W1_0, W2_0, W3_0 - W3_2], axis=1)
    wb = jnp.concatenate([z4, W2_1, W3_1], axis=1)
    wc = jnp.concatenate([z4, z4, 2.0 * W3_2], axis=1)
    bias = jnp.concatenate([b1, b2, b3])[None, :]
    return _matmul(x, t1, p2[0], p2[1], wa, wb, wc, bias)


# speedup vs baseline: 16.9151x; 1.1954x over previous
import functools

import jax
import jax.numpy as jnp
from jax import lax
from jax.experimental import pallas as pl
from jax.experimental.pallas import tpu as pltpu
from jax.experimental.pallas import tpu_sc as plsc

N = 10000
E = 320000
D = 128
DO3 = 1200

NC = 2
NS = 16
NW = NC * NS

EA = E // NS
EW = E // NW
WIN = 80
NWIN = EW // WIN

_MESH = plsc.VectorSubcoreMesh(core_axis_name="c", subcore_axis_name="s")


def _rsqrt16(d):
    di = lax.bitcast_convert_type(d, jnp.int32)
    yi = jnp.int32(0x5F3759DF) - lax.shift_right_logical(di, 1)
    y = lax.bitcast_convert_type(yi, jnp.float32)
    for _ in range(3):
        y = y * (1.5 - 0.5 * d * y * y)
    return y


def _off_body(row_hbm, col_hbm, w_hbm, off_hbm, deg_sh, ibuf, jbuf, wbuf, abuf):
    c = lax.axis_index("c")
    s = lax.axis_index("s")
    wid = s * NC + c

    e0 = s * EA
    pltpu.sync_copy(row_hbm.at[pl.ds(e0, EA)], ibuf)
    pltpu.sync_copy(col_hbm.at[pl.ds(e0, EA)], jbuf)
    pltpu.sync_copy(w_hbm.at[pl.ds(e0, EA)], wbuf)

    def _z(i, _):
        abuf[pl.ds(i * 16, 16)] = jnp.zeros((16,), jnp.float32)
        return _
    lax.fori_loop(0, 40, _z, 0)
    pltpu.sync_copy(abuf.at[pl.ds(0, 640)], deg_sh.at[pl.ds(s * 640, 640)])

    def _wm(i, _):
        sl = pl.ds(i * 16, 16)
        r = ibuf[sl]
        cc = jbuf[sl]
        w = wbuf[sl]
        abuf[sl] = jnp.where(r == cc, 0.0, w)
        return _
    lax.fori_loop(0, EA // 16, _wm, 0)

    plsc.subcore_barrier()
    pltpu.sync_copy(abuf, deg_sh.at[ibuf], add=True)
    plsc.subcore_barrier()

    pltpu.sync_copy(deg_sh.at[pl.ds(0, N)], abuf.at[pl.ds(0, N)])

    def _ns(i, _):
        sl = pl.ds(i * 16, 16)
        d = abuf[sl]
        y = _rsqrt16(d)
        abuf[sl] = jnp.where(d > 0.0, y, 0.0)
        return _
    lax.fori_loop(0, N // 16, _ns, 0)

    b0 = wid * EW
    pltpu.sync_copy(row_hbm.at[pl.ds(b0, EW)], ibuf.at[pl.ds(0, EW)])
    pltpu.sync_copy(col_hbm.at[pl.ds(b0, EW)], jbuf.at[pl.ds(0, EW)])
    pltpu.sync_copy(w_hbm.at[pl.ds(b0, EW)], wbuf.at[pl.ds(0, EW)])

    def _off(i, _):
        sl = pl.ds(i * 16, 16)
        r = ibuf[sl]
        cc = jbuf[sl]
        w = wbuf[sl]
        dr = plsc.load_gather(abuf, [r])
        dc = plsc.load_gather(abuf, [cc])
        v = -(dr * w * dc)
        wbuf[pl.ds(EW + i * 16, 16)] = jnp.where(r == cc, 0.0, v)
        return _
    lax.fori_loop(0, EW // 16, _off, 0)

    pltpu.sync_copy(wbuf.at[pl.ds(EW, EW)], off_hbm.at[pl.ds(b0, EW)])


_off_kernel = functools.partial(
    pl.kernel,
    out_type=jax.ShapeDtypeStruct((E,), jnp.float32),
    mesh=_MESH,
    compiler_params=pltpu.CompilerParams(needs_layout_passes=False, use_tc_tiling_on_sc=False),
    scratch_types=[
        pltpu.VMEM_SHARED((10240,), jnp.float32),
        pltpu.VMEM((EA,), jnp.int32),
        pltpu.VMEM((EA,), jnp.int32),
        pltpu.VMEM((2 * EW,), jnp.float32),
        pltpu.VMEM((2 * EW,), jnp.float32),
    ],
)(_off_body)


NG = 4
NE = 8


def _prop_body(v_hbm, e3_hbm, out_hbm,
               acc_sh, gbuf, ebuf, zbuf, gsem, ssem, esem):
    c = lax.axis_index("c")
    s = lax.axis_index("s")
    wid = s * NC + c

    for r in range(8):
        for f in range(8):
            zbuf[r, pl.ds(f * 16, 16)] = jnp.zeros((16,), jnp.float32)

    def _za(k, _):
        pltpu.sync_copy(zbuf, acc_sh.at[pl.ds(s * 625 + k * 8, 8)])
        return _
    lax.fori_loop(0, 78, _za, 0)
    pltpu.sync_copy(zbuf.at[pl.ds(0, 1)], acc_sh.at[pl.ds(s * 625 + 624, 1)])
    plsc.subcore_barrier()

    g0 = wid * NWIN

    def _start_ebuf(w, v):
        pltpu.async_copy(e3_hbm.at[g0 + w], ebuf.at[v], esem.at[v])

    def _wait_ebuf(v):
        pltpu.make_async_copy(e3_hbm.at[0], ebuf.at[v], esem.at[v]).wait()

    def _start_gather(w, u, v):
        pltpu.async_copy(v_hbm.at[ebuf.at[v, 0]], gbuf.at[u], gsem.at[u])

    def _wait_gather(u):
        pltpu.make_async_copy(v_hbm.at[pl.ds(0, WIN)], gbuf.at[u],
                              gsem.at[u]).wait()

    def _scale(u, v):
        off_ref = ebuf.at[v, 2]

        def _edge4(e4, _2):
            for q in range(4):
                e = e4 * 4 + q
                bits = plsc.load_gather(
                    off_ref, [jnp.full((16,), 0, jnp.int32) + e])
                splat = plsc.bitcast(bits, jnp.float32)
                for f in range(8):
                    sl = pl.ds(f * 16, 16)
                    gbuf[u, e, sl] = gbuf[u, e, sl] * splat
            return _2
        lax.fori_loop(0, WIN // 4, _edge4, 0)

    def _start_scatter(u, v):
        pltpu.async_copy(gbuf.at[u], acc_sh.at[ebuf.at[v, 1]], ssem.at[u],
                         add=True)

    def _wait_scatter(u):
        pltpu.make_async_copy(v_hbm.at[pl.ds(0, WIN)], gbuf.at[u],
                              ssem.at[u]).wait()

    def _launch(t, w2):
        if w2 >= NG:
            _wait_scatter(w2 % NG)
        if t + 6 <= NWIN - 1:
            _start_ebuf(t + 6, (t + 6) % NE)
        _wait_ebuf(w2 % NE)
        _start_gather(w2, w2 % NG, w2 % NE)

    def _compute(t):
        u, v = t % NG, t % NE
        _wait_gather(u)
        _scale(u, v)
        _start_scatter(u, v)

    for w in range(6):
        _start_ebuf(w, w)
    for w in range(2):
        _wait_ebuf(w)
        _start_gather(w, w, w)

    for t in range(8):
        _launch(t, t + 2)
        _compute(t)

    def _octet(k, _):
        t0 = k * 8
        for j in range(8):
            tj = t0 + j
            if True:
                _wait_scatter((j + 2) % NG)
                pltpu.async_copy(e3_hbm.at[g0 + tj + 6], ebuf.at[(j + 6) % NE],
                                 esem.at[(j + 6) % NE])
                _wait_ebuf((j + 2) % NE)
                pltpu.async_copy(v_hbm.at[ebuf.at[(j + 2) % NE, 0]],
                                 gbuf.at[(j + 2) % NG], gsem.at[(j + 2) % NG])
                _wait_gather(j % NG)
                _scale(j % NG, j % NE)
                _start_scatter(j % NG, j % NE)
        return _
    lax.fori_loop(1, 14, _octet, 0)

    for t in range(112, 120):
        _launch(t, t + 2)
        _compute(t)
    for t in range(120, 125):
        if t + 2 <= NWIN - 1:
            _launch(t, t + 2)
        _compute(t)

    for u in range(NG):
        _wait_scatter(u)

    plsc.subcore_barrier()
    pltpu.sync_copy(acc_sh.at[pl.ds(s * 625, 625)],
                    out_hbm.at[c, pl.ds(s * 625, 625)])


_prop_kernel = functools.partial(
    pl.kernel,
    out_type=jax.ShapeDtypeStruct((NC, N, D), jnp.float32),
    mesh=_MESH,
    compiler_params=pltpu.CompilerParams(needs_layout_passes=False, use_tc_tiling_on_sc=False),
    scratch_types=[
        pltpu.VMEM_SHARED((N, D), jnp.float32),
        pltpu.VMEM((NG, WIN, D), jnp.float32),
        pltpu.VMEM((NE, 3, WIN), jnp.int32),
        pltpu.VMEM((8, D), jnp.float32),
        pltpu.SemaphoreType.DMA((NG,)),
        pltpu.SemaphoreType.DMA((NG,)),
        pltpu.SemaphoreType.DMA((NE,)),
    ],
)(_prop_body)


def _add_body(a_ref, b_ref, o_ref):
    o_ref[...] = a_ref[...] + b_ref[...]


def _combine_partials(p):
    return pl.pallas_call(
        _add_body,
        grid=(10,),
        in_specs=[pl.BlockSpec((N // 10, D), lambda i: (i, 0)),
                  pl.BlockSpec((N // 10, D), lambda i: (i, 0))],
        out_specs=pl.BlockSpec((N // 10, D), lambda i: (i, 0)),
        out_shape=jax.ShapeDtypeStruct((N, D), jnp.float32),
    )(p[0], p[1])


def _mm_body(x_ref, t1_ref, q0_ref, q1_ref, wa_ref, wb_ref, wc_ref, b_ref,
             o_ref):
    xb = x_ref[...]
    t1 = t1_ref[...]
    qb = q0_ref[...] + q1_ref[...]
    o = jnp.dot(xb, wa_ref[...], preferred_element_type=jnp.float32)
    o += jnp.dot(t1, wb_ref[...], preferred_element_type=jnp.float32)
    o += jnp.dot(qb, wc_ref[...], preferred_element_type=jnp.float32)
    o_ref[...] = o + b_ref[...]


def _matmul(x, t1, q0, q1, wa, wb, wc, bias):
    br = N // 10
    return pl.pallas_call(
        _mm_body,
        grid=(10,),
        in_specs=[pl.BlockSpec((br, D), lambda i: (i, 0)),
                  pl.BlockSpec((br, D), lambda i: (i, 0)),
                  pl.BlockSpec((br, D), lambda i: (i, 0)),
                  pl.BlockSpec((br, D), lambda i: (i, 0)),
                  pl.BlockSpec((D, DO3), lambda i: (0, 0)),
                  pl.BlockSpec((D, DO3), lambda i: (0, 0)),
                  pl.BlockSpec((D, DO3), lambda i: (0, 0)),
                  pl.BlockSpec((1, DO3), lambda i: (0, 0))],
        out_specs=pl.BlockSpec((br, DO3), lambda i: (i, 0)),
        out_shape=jax.ShapeDtypeStruct((N, DO3), jnp.float32),
    )(x, t1, q0, q1, wa, wb, wc, bias)


def kernel(x, edge_index, edge_weight, W1_0, b1, W2_0, W2_1, b2,
           W3_0, W3_1, W3_2, b3):
    row = edge_index[0]
    col = edge_index[1]

    off = _off_kernel(row, col, edge_weight)
    e3 = jnp.stack([row.reshape(NW * NWIN, WIN),
                    col.reshape(NW * NWIN, WIN),
                    lax.bitcast_convert_type(off, jnp.int32)
                       .reshape(NW * NWIN, WIN)], axis=1)
    p1 = _prop_kernel(x, e3)
    t1 = _combine_partials(p1)
    p2 = _prop_kernel(t1, e3)

    z4 = jnp.zeros((D, 400), jnp.float32)
    wa = jnp.concatenate([W1_0, W2_0, W3_0 - W3_2], axis=1)
    wb = jnp.concatenate([z4, W2_1, W3_1], axis=1)
    wc = jnp.concatenate([z4, z4, 2.0 * W3_2], axis=1)
    bias = jnp.concatenate([b1, b2, b3])[None, :]
    return _matmul(x, t1, p2[0], p2[1], wa, wb, wc, bias)


# speedup vs baseline: 17.5299x; 1.0363x over previous
import functools

import jax
import jax.numpy as jnp
from jax import lax
from jax.experimental import pallas as pl
from jax.experimental.pallas import tpu as pltpu
from jax.experimental.pallas import tpu_sc as plsc

N = 10000
E = 320000
D = 128
DO3 = 1200

NC = 2
NS = 16
NW = NC * NS

EA = E // NS
EW = E // NW
WIN = 80
NWIN = EW // WIN

_MESH = plsc.VectorSubcoreMesh(core_axis_name="c", subcore_axis_name="s")


def _rsqrt16(d):
    di = lax.bitcast_convert_type(d, jnp.int32)
    yi = jnp.int32(0x5F3759DF) - lax.shift_right_logical(di, 1)
    y = lax.bitcast_convert_type(yi, jnp.float32)
    for _ in range(3):
        y = y * (1.5 - 0.5 * d * y * y)
    return y


def _off_body(row_hbm, col_hbm, w_hbm, off_hbm, deg_sh, ibuf, jbuf, wbuf, abuf):
    c = lax.axis_index("c")
    s = lax.axis_index("s")
    wid = s * NC + c

    e0 = s * EA
    pltpu.sync_copy(row_hbm.at[pl.ds(e0, EA)], ibuf)
    pltpu.sync_copy(col_hbm.at[pl.ds(e0, EA)], jbuf)
    pltpu.sync_copy(w_hbm.at[pl.ds(e0, EA)], wbuf)

    def _z(i, _):
        abuf[pl.ds(i * 16, 16)] = jnp.zeros((16,), jnp.float32)
        return _
    lax.fori_loop(0, 40, _z, 0)
    pltpu.sync_copy(abuf.at[pl.ds(0, 640)], deg_sh.at[pl.ds(s * 640, 640)])

    def _wm(i, _):
        sl = pl.ds(i * 16, 16)
        r = ibuf[sl]
        cc = jbuf[sl]
        w = wbuf[sl]
        abuf[sl] = jnp.where(r == cc, 0.0, w)
        return _
    lax.fori_loop(0, EA // 16, _wm, 0)

    plsc.subcore_barrier()
    pltpu.sync_copy(abuf, deg_sh.at[ibuf], add=True)
    plsc.subcore_barrier()

    pltpu.sync_copy(deg_sh.at[pl.ds(0, N)], abuf.at[pl.ds(0, N)])

    def _ns(i, _):
        sl = pl.ds(i * 16, 16)
        d = abuf[sl]
        y = _rsqrt16(d)
        abuf[sl] = jnp.where(d > 0.0, y, 0.0)
        return _
    lax.fori_loop(0, N // 16, _ns, 0)

    b0 = wid * EW
    pltpu.sync_copy(row_hbm.at[pl.ds(b0, EW)], ibuf.at[pl.ds(0, EW)])
    pltpu.sync_copy(col_hbm.at[pl.ds(b0, EW)], jbuf.at[pl.ds(0, EW)])
    pltpu.sync_copy(w_hbm.at[pl.ds(b0, EW)], wbuf.at[pl.ds(0, EW)])

    def _off(i, _):
        sl = pl.ds(i * 16, 16)
        r = ibuf[sl]
        cc = jbuf[sl]
        w = wbuf[sl]
        dr = plsc.load_gather(abuf, [r])
        dc = plsc.load_gather(abuf, [cc])
        v = -(dr * w * dc)
        wbuf[pl.ds(EW + i * 16, 16)] = jnp.where(r == cc, 0.0, v)
        return _
    lax.fori_loop(0, EW // 16, _off, 0)

    pltpu.sync_copy(wbuf.at[pl.ds(EW, EW)], off_hbm.at[pl.ds(b0, EW)])


_off_kernel = functools.partial(
    pl.kernel,
    out_type=jax.ShapeDtypeStruct((E,), jnp.float32),
    mesh=_MESH,
    compiler_params=pltpu.CompilerParams(needs_layout_passes=False, use_tc_tiling_on_sc=False),
    scratch_types=[
        pltpu.VMEM_SHARED((10240,), jnp.float32),
        pltpu.VMEM((EA,), jnp.int32),
        pltpu.VMEM((EA,), jnp.int32),
        pltpu.VMEM((2 * EW,), jnp.float32),
        pltpu.VMEM((2 * EW,), jnp.float32),
    ],
)(_off_body)


NG = 4
NE = 8


def _prop_body(v_hbm, e3_hbm, out_hbm,
               acc_sh, gbuf, ebuf, zbuf, gsem, ssem, esem):
    c = lax.axis_index("c")
    s = lax.axis_index("s")
    wid = s * NC + c

    for r in range(8):
        for f in range(4):
            zbuf[r, pl.ds(f * 32, 32)] = jnp.zeros((32,), jnp.bfloat16)

    def _za(k, _):
        pltpu.sync_copy(zbuf, acc_sh.at[pl.ds(s * 625 + k * 8, 8)])
        return _
    lax.fori_loop(0, 78, _za, 0)
    pltpu.sync_copy(zbuf.at[pl.ds(0, 1)], acc_sh.at[pl.ds(s * 625 + 624, 1)])
    plsc.subcore_barrier()

    g0 = wid * NWIN

    def _start_ebuf(w, v):
        pltpu.async_copy(e3_hbm.at[g0 + w], ebuf.at[v], esem.at[v])

    def _wait_ebuf(v):
        pltpu.make_async_copy(e3_hbm.at[0], ebuf.at[v], esem.at[v]).wait()

    def _start_gather(w, u, v):
        pltpu.async_copy(v_hbm.at[ebuf.at[v, 0]], gbuf.at[u], gsem.at[u])

    def _wait_gather(u):
        pltpu.make_async_copy(v_hbm.at[pl.ds(0, WIN)], gbuf.at[u],
                              gsem.at[u]).wait()

    def _scale(u, v):
        off_ref = ebuf.at[v, 2]

        def _edge4(e4, _2):
            for q in range(4):
                e = e4 * 4 + q
                bits = plsc.load_gather(
                    off_ref, [jnp.full((16,), 0, jnp.int32) + e])
                offf = plsc.bitcast(bits, jnp.float32)
                splat = plsc.pack(offf, offf,
                                  format=plsc.PackFormat.INTERLEAVED)
                for f in range(4):
                    sl = pl.ds(f * 32, 32)
                    gbuf[u, e, sl] = gbuf[u, e, sl] * splat
            return _2
        lax.fori_loop(0, WIN // 4, _edge4, 0)

    def _start_scatter(u, v):
        pltpu.async_copy(gbuf.at[u], acc_sh.at[ebuf.at[v, 1]], ssem.at[u],
                         add=True)

    def _wait_scatter(u):
        pltpu.make_async_copy(v_hbm.at[pl.ds(0, WIN)], gbuf.at[u],
                              ssem.at[u]).wait()

    def _launch(t, w2):
        if w2 >= NG:
            _wait_scatter(w2 % NG)
        if t + 6 <= NWIN - 1:
            _start_ebuf(t + 6, (t + 6) % NE)
        _wait_ebuf(w2 % NE)
        _start_gather(w2, w2 % NG, w2 % NE)

    def _compute(t):
        u, v = t % NG, t % NE
        _wait_gather(u)
        _scale(u, v)
        _start_scatter(u, v)

    for w in range(6):
        _start_ebuf(w, w)
    for w in range(2):
        _wait_ebuf(w)
        _start_gather(w, w, w)

    for t in range(8):
        _launch(t, t + 2)
        _compute(t)

    def _octet(k, _):
        t0 = k * 8
        for j in range(8):
            tj = t0 + j
            if True:
                _wait_scatter((j + 2) % NG)
                pltpu.async_copy(e3_hbm.at[g0 + tj + 6], ebuf.at[(j + 6) % NE],
                                 esem.at[(j + 6) % NE])
                _wait_ebuf((j + 2) % NE)
                pltpu.async_copy(v_hbm.at[ebuf.at[(j + 2) % NE, 0]],
                                 gbuf.at[(j + 2) % NG], gsem.at[(j + 2) % NG])
                _wait_gather(j % NG)
                _scale(j % NG, j % NE)
                _start_scatter(j % NG, j % NE)
        return _
    lax.fori_loop(1, 14, _octet, 0)

    for t in range(112, 120):
        _launch(t, t + 2)
        _compute(t)
    for t in range(120, 125):
        if t + 2 <= NWIN - 1:
            _launch(t, t + 2)
        _compute(t)

    for u in range(NG):
        _wait_scatter(u)

    plsc.subcore_barrier()
    pltpu.sync_copy(acc_sh.at[pl.ds(s * 625, 625)],
                    out_hbm.at[c, pl.ds(s * 625, 625)])


_prop_kernel = functools.partial(
    pl.kernel,
    out_type=jax.ShapeDtypeStruct((NC, N, D), jnp.bfloat16),
    mesh=_MESH,
    compiler_params=pltpu.CompilerParams(needs_layout_passes=False, use_tc_tiling_on_sc=False),
    scratch_types=[
        pltpu.VMEM_SHARED((N, D), jnp.bfloat16),
        pltpu.VMEM((NG, WIN, D), jnp.bfloat16),
        pltpu.VMEM((NE, 3, WIN), jnp.int32),
        pltpu.VMEM((8, D), jnp.bfloat16),
        pltpu.SemaphoreType.DMA((NG,)),
        pltpu.SemaphoreType.DMA((NG,)),
        pltpu.SemaphoreType.DMA((NE,)),
    ],
)(_prop_body)


def _add_body(a_ref, b_ref, o_ref):
    o_ref[...] = (a_ref[...].astype(jnp.float32)
                  + b_ref[...].astype(jnp.float32)).astype(jnp.bfloat16)


def _combine_partials(p):
    return pl.pallas_call(
        _add_body,
        grid=(10,),
        in_specs=[pl.BlockSpec((N // 10, D), lambda i: (i, 0)),
                  pl.BlockSpec((N // 10, D), lambda i: (i, 0))],
        out_specs=pl.BlockSpec((N // 10, D), lambda i: (i, 0)),
        out_shape=jax.ShapeDtypeStruct((N, D), jnp.bfloat16),
    )(p[0], p[1])


def _mm_body(x_ref, t1_ref, q0_ref, q1_ref, wa_ref, wb_ref, wc_ref, b_ref,
             o_ref):
    xb = x_ref[...]
    t1 = t1_ref[...].astype(jnp.float32)
    qb = (q0_ref[...].astype(jnp.float32)
          + q1_ref[...].astype(jnp.float32))
    o = jnp.dot(xb, wa_ref[...], preferred_element_type=jnp.float32)
    o += jnp.dot(t1, wb_ref[...], preferred_element_type=jnp.float32)
    o += jnp.dot(qb, wc_ref[...], preferred_element_type=jnp.float32)
    o_ref[...] = o + b_ref[...]


def _matmul(x, t1, q0, q1, wa, wb, wc, bias):
    br = N // 10
    return pl.pallas_call(
        _mm_body,
        grid=(10,),
        in_specs=[pl.BlockSpec((br, D), lambda i: (i, 0)),
                  pl.BlockSpec((br, D), lambda i: (i, 0)),
                  pl.BlockSpec((br, D), lambda i: (i, 0)),
                  pl.BlockSpec((br, D), lambda i: (i, 0)),
                  pl.BlockSpec((D, DO3), lambda i: (0, 0)),
                  pl.BlockSpec((D, DO3), lambda i: (0, 0)),
                  pl.BlockSpec((D, DO3), lambda i: (0, 0)),
                  pl.BlockSpec((1, DO3), lambda i: (0, 0))],
        out_specs=pl.BlockSpec((br, DO3), lambda i: (i, 0)),
        out_shape=jax.ShapeDtypeStruct((N, DO3), jnp.float32),
    )(x, t1, q0, q1, wa, wb, wc, bias)


def kernel(x, edge_index, edge_weight, W1_0, b1, W2_0, W2_1, b2,
           W3_0, W3_1, W3_2, b3):
    row = edge_index[0]
    col = edge_index[1]

    off = _off_kernel(row, col, edge_weight)
    e3 = jnp.stack([row.reshape(NW * NWIN, WIN),
                    col.reshape(NW * NWIN, WIN),
                    lax.bitcast_convert_type(off, jnp.int32)
                       .reshape(NW * NWIN, WIN)], axis=1)
    p1 = _prop_kernel(x.astype(jnp.bfloat16), e3)
    t1 = _combine_partials(p1)
    p2 = _prop_kernel(t1, e3)

    z4 = jnp.zeros((D, 400), jnp.float32)
    wa = jnp.concatenate([W1_0, W2_0, W3_0 - W3_2], axis=1)
    wb = jnp.concatenate([z4, W2_1, W3_1], axis=1)
    wc = jnp.concatenate([z4, z4, 2.0 * W3_2], axis=1)
    bias = jnp.concatenate([b1, b2, b3])[None, :]
    return _matmul(x, t1, p2[0], p2[1], wa, wb, wc, bias)


# speedup vs baseline: 17.7034x; 1.0099x over previous
import functools

import jax
import jax.numpy as jnp
from jax import lax
from jax.experimental import pallas as pl
from jax.experimental.pallas import tpu as pltpu
from jax.experimental.pallas import tpu_sc as plsc

N = 10000
E = 320000
D = 128
DO3 = 1200

NC = 2
NS = 16
NW = NC * NS

EA = E // NS
EW = E // NW
WIN = 80
NWIN = EW // WIN

_MESH = plsc.VectorSubcoreMesh(core_axis_name="c", subcore_axis_name="s")


def _rsqrt16(d):
    di = lax.bitcast_convert_type(d, jnp.int32)
    yi = jnp.int32(0x5F3759DF) - lax.shift_right_logical(di, 1)
    y = lax.bitcast_convert_type(yi, jnp.float32)
    for _ in range(3):
        y = y * (1.5 - 0.5 * d * y * y)
    return y


def _off_body(row_hbm, col_hbm, w_hbm, off_hbm, deg_sh, ibuf, jbuf, wbuf, abuf):
    c = lax.axis_index("c")
    s = lax.axis_index("s")
    wid = s * NC + c

    e0 = s * EA
    pltpu.sync_copy(row_hbm.at[pl.ds(e0, EA)], ibuf)
    pltpu.sync_copy(col_hbm.at[pl.ds(e0, EA)], jbuf)
    pltpu.sync_copy(w_hbm.at[pl.ds(e0, EA)], wbuf)

    def _z(i, _):
        abuf[pl.ds(i * 16, 16)] = jnp.zeros((16,), jnp.float32)
        return _
    lax.fori_loop(0, 40, _z, 0)
    pltpu.sync_copy(abuf.at[pl.ds(0, 640)], deg_sh.at[pl.ds(s * 640, 640)])

    def _wm(i, _):
        sl = pl.ds(i * 16, 16)
        r = ibuf[sl]
        cc = jbuf[sl]
        w = wbuf[sl]
        abuf[sl] = jnp.where(r == cc, 0.0, w)
        return _
    lax.fori_loop(0, EA // 16, _wm, 0)

    plsc.subcore_barrier()
    pltpu.sync_copy(abuf, deg_sh.at[ibuf], add=True)
    plsc.subcore_barrier()

    pltpu.sync_copy(deg_sh.at[pl.ds(0, N)], abuf.at[pl.ds(0, N)])

    def _ns(i, _):
        sl = pl.ds(i * 16, 16)
        d = abuf[sl]
        y = _rsqrt16(d)
        abuf[sl] = jnp.where(d > 0.0, y, 0.0)
        return _
    lax.fori_loop(0, N // 16, _ns, 0)

    b0 = wid * EW
    pltpu.sync_copy(row_hbm.at[pl.ds(b0, EW)], ibuf.at[pl.ds(0, EW)])
    pltpu.sync_copy(col_hbm.at[pl.ds(b0, EW)], jbuf.at[pl.ds(0, EW)])
    pltpu.sync_copy(w_hbm.at[pl.ds(b0, EW)], wbuf.at[pl.ds(0, EW)])

    def _off(i, _):
        sl = pl.ds(i * 16, 16)
        r = ibuf[sl]
        cc = jbuf[sl]
        w = wbuf[sl]
        dr = plsc.load_gather(abuf, [r])
        dc = plsc.load_gather(abuf, [cc])
        v = -(dr * w * dc)
        wbuf[pl.ds(EW + i * 16, 16)] = jnp.where(r == cc, 0.0, v)
        return _
    lax.fori_loop(0, EW // 16, _off, 0)

    pltpu.sync_copy(wbuf.at[pl.ds(EW, EW)], off_hbm.at[pl.ds(b0, EW)])


_off_kernel = functools.partial(
    pl.kernel,
    out_type=jax.ShapeDtypeStruct((E,), jnp.float32),
    mesh=_MESH,
    compiler_params=pltpu.CompilerParams(needs_layout_passes=False, use_tc_tiling_on_sc=False),
    scratch_types=[
        pltpu.VMEM_SHARED((10240,), jnp.float32),
        pltpu.VMEM((EA,), jnp.int32),
        pltpu.VMEM((EA,), jnp.int32),
        pltpu.VMEM((2 * EW,), jnp.float32),
        pltpu.VMEM((2 * EW,), jnp.float32),
    ],
)(_off_body)


NG = 8
NE = 8


def _prop_body(v_hbm, e3_hbm, out_hbm,
               acc_sh, gbuf, ebuf, zbuf, gsem, ssem, esem):
    c = lax.axis_index("c")
    s = lax.axis_index("s")
    wid = s * NC + c

    for r in range(8):
        for f in range(4):
            zbuf[r, pl.ds(f * 32, 32)] = jnp.zeros((32,), jnp.bfloat16)

    def _za(k, _):
        pltpu.sync_copy(zbuf, acc_sh.at[pl.ds(s * 625 + k * 8, 8)])
        return _
    lax.fori_loop(0, 78, _za, 0)
    pltpu.sync_copy(zbuf.at[pl.ds(0, 1)], acc_sh.at[pl.ds(s * 625 + 624, 1)])
    plsc.subcore_barrier()

    g0 = wid * NWIN

    def _start_ebuf(w, v):
        pltpu.async_copy(e3_hbm.at[g0 + w], ebuf.at[v], esem.at[v])

    def _wait_ebuf(v):
        pltpu.make_async_copy(e3_hbm.at[0], ebuf.at[v], esem.at[v]).wait()

    def _start_gather(w, u, v):
        pltpu.async_copy(v_hbm.at[ebuf.at[v, 0]], gbuf.at[u], gsem.at[u])

    def _wait_gather(u):
        pltpu.make_async_copy(v_hbm.at[pl.ds(0, WIN)], gbuf.at[u],
                              gsem.at[u]).wait()

    def _scale(u, v):
        off_ref = ebuf.at[v, 2]

        def _edge4(e4, _2):
            for q in range(4):
                e = e4 * 4 + q
                bits = plsc.load_gather(
                    off_ref, [jnp.full((16,), 0, jnp.int32) + e])
                offf = plsc.bitcast(bits, jnp.float32)
                splat = plsc.pack(offf, offf,
                                  format=plsc.PackFormat.INTERLEAVED)
                for f in range(4):
                    sl = pl.ds(f * 32, 32)
                    gbuf[u, e, sl] = gbuf[u, e, sl] * splat
            return _2
        lax.fori_loop(0, WIN // 4, _edge4, 0)

    def _start_scatter(u, v):
        pltpu.async_copy(gbuf.at[u], acc_sh.at[ebuf.at[v, 1]], ssem.at[u],
                         add=True)

    def _wait_scatter(u):
        pltpu.make_async_copy(v_hbm.at[pl.ds(0, WIN)], gbuf.at[u],
                              ssem.at[u]).wait()

    def _launch(t, w2):
        if w2 >= NG:
            _wait_scatter(w2 % NG)
        if t + 6 <= NWIN - 1:
            _start_ebuf(t + 6, (t + 6) % NE)
        _wait_ebuf(w2 % NE)
        _start_gather(w2, w2 % NG, w2 % NE)

    def _compute(t):
        u, v = t % NG, t % NE
        _wait_gather(u)
        _scale(u, v)
        _start_scatter(u, v)

    for w in range(6):
        _start_ebuf(w, w)
    for w in range(2):
        _wait_ebuf(w)
        _start_gather(w, w, w)

    for t in range(8):
        _launch(t, t + 2)
        _compute(t)

    def _octet(k, _):
        t0 = k * 8
        for j in range(8):
            tj = t0 + j
            if True:
                _wait_scatter((j + 2) % NG)
                pltpu.async_copy(e3_hbm.at[g0 + tj + 6], ebuf.at[(j + 6) % NE],
                                 esem.at[(j + 6) % NE])
                _wait_ebuf((j + 2) % NE)
                pltpu.async_copy(v_hbm.at[ebuf.at[(j + 2) % NE, 0]],
                                 gbuf.at[(j + 2) % NG], gsem.at[(j + 2) % NG])
                _wait_gather(j % NG)
                _scale(j % NG, j % NE)
                _start_scatter(j % NG, j % NE)
        return _
    lax.fori_loop(1, 14, _octet, 0)

    for t in range(112, 120):
        _launch(t, t + 2)
        _compute(t)
    for t in range(120, 125):
        if t + 2 <= NWIN - 1:
            _launch(t, t + 2)
        _compute(t)

    for u in range(NG):
        _wait_scatter(u)

    plsc.subcore_barrier()
    pltpu.sync_copy(acc_sh.at[pl.ds(s * 625, 625)],
                    out_hbm.at[c, pl.ds(s * 625, 625)])


_prop_kernel = functools.partial(
    pl.kernel,
    out_type=jax.ShapeDtypeStruct((NC, N, D), jnp.bfloat16),
    mesh=_MESH,
    compiler_params=pltpu.CompilerParams(needs_layout_passes=False, use_tc_tiling_on_sc=False),
    scratch_types=[
        pltpu.VMEM_SHARED((N, D), jnp.bfloat16),
        pltpu.VMEM((NG, WIN, D), jnp.bfloat16),
        pltpu.VMEM((NE, 3, WIN), jnp.int32),
        pltpu.VMEM((8, D), jnp.bfloat16),
        pltpu.SemaphoreType.DMA((NG,)),
        pltpu.SemaphoreType.DMA((NG,)),
        pltpu.SemaphoreType.DMA((NE,)),
    ],
)(_prop_body)


def _add_body(a_ref, b_ref, o_ref):
    o_ref[...] = (a_ref[...].astype(jnp.float32)
                  + b_ref[...].astype(jnp.float32)).astype(jnp.bfloat16)


def _combine_partials(p):
    return pl.pallas_call(
        _add_body,
        grid=(10,),
        in_specs=[pl.BlockSpec((N // 10, D), lambda i: (i, 0)),
                  pl.BlockSpec((N // 10, D), lambda i: (i, 0))],
        out_specs=pl.BlockSpec((N // 10, D), lambda i: (i, 0)),
        out_shape=jax.ShapeDtypeStruct((N, D), jnp.bfloat16),
    )(p[0], p[1])


def _mm_body(x_ref, t1_ref, q0_ref, q1_ref, wa_ref, wb_ref, wc_ref, b_ref,
             o_ref):
    xb = x_ref[...]
    t1 = t1_ref[...]
    qb = q0_ref[...] + q1_ref[...]
    o = jnp.dot(xb, wa_ref[...], preferred_element_type=jnp.float32)
    o += jnp.dot(t1, wb_ref[...], preferred_element_type=jnp.float32)
    o += jnp.dot(qb, wc_ref[...], preferred_element_type=jnp.float32)
    o_ref[...] = o + b_ref[...]


def _matmul(x, t1, q0, q1, wa, wb, wc, bias):
    br = N // 10
    return pl.pallas_call(
        _mm_body,
        grid=(10,),
        in_specs=[pl.BlockSpec((br, D), lambda i: (i, 0)),
                  pl.BlockSpec((br, D), lambda i: (i, 0)),
                  pl.BlockSpec((br, D), lambda i: (i, 0)),
                  pl.BlockSpec((br, D), lambda i: (i, 0)),
                  pl.BlockSpec((D, DO3), lambda i: (0, 0)),
                  pl.BlockSpec((D, DO3), lambda i: (0, 0)),
                  pl.BlockSpec((D, DO3), lambda i: (0, 0)),
                  pl.BlockSpec((1, DO3), lambda i: (0, 0))],
        out_specs=pl.BlockSpec((br, DO3), lambda i: (i, 0)),
        out_shape=jax.ShapeDtypeStruct((N, DO3), jnp.float32),
    )(x, t1, q0, q1, wa, wb, wc, bias)


def kernel(x, edge_index, edge_weight, W1_0, b1, W2_0, W2_1, b2,
           W3_0, W3_1, W3_2, b3):
    row = edge_index[0]
    col = edge_index[1]

    off = _off_kernel(row, col, edge_weight)
    e3 = jnp.stack([row.reshape(NW * NWIN, WIN),
                    col.reshape(NW * NWIN, WIN),
                    lax.bitcast_convert_type(off, jnp.int32)
                       .reshape(NW * NWIN, WIN)], axis=1)
    p1 = _prop_kernel(x.astype(jnp.bfloat16), e3)
    t1 = _combine_partials(p1)
    p2 = _prop_kernel(t1, e3)

    z4 = jnp.zeros((D, 400), jnp.float32)
    wa = jnp.concatenate([W1_0, W2_0, W3_0 - W3_2], axis=1)
    wb = jnp.concatenate([z4, W2_1, W3_1], axis=1).astype(jnp.bfloat16)
    wc = jnp.concatenate([z4, z4, 2.0 * W3_2], axis=1).astype(jnp.bfloat16)
    bias = jnp.concatenate([b1, b2, b3])[None, :]
    return _matmul(x, t1, p2[0], p2[1], wa, wb, wc, bias)
